# baseline (speedup 1.0000x reference)
import functools

import jax
import jax.numpy as jnp
from jax import lax
from jax.experimental import pallas as pl
from jax.experimental.pallas import tpu as pltpu
from jax.experimental.pallas import tpu_sc as plsc

L = 16
NC = 2
NS = 16
NW = NC * NS
BATCH = 16384
D = 32
B_PER_W = BATCH // NW


def _mf_body(uids_hbm, mids_hbm, utab_hbm, mtab_hbm, out_hbm,
             uidx_v, midx_v, urows_v, mrows_v, outb_v, usem, msem):
    wid = lax.axis_index("s") * NC + lax.axis_index("c")
    base = wid * B_PER_W

    pltpu.sync_copy(uids_hbm.at[pl.ds(base, B_PER_W)], uidx_v)
    pltpu.sync_copy(mids_hbm.at[pl.ds(base, B_PER_W)], midx_v)
    cu = pltpu.async_copy(utab_hbm.at[uidx_v], urows_v, usem)
    cm = pltpu.async_copy(mtab_hbm.at[midx_v], mrows_v, msem)
    cu.wait()
    cm.wait()

    def body(g, carry):
        rows = g * L + lax.iota(jnp.int32, L)
        acc = jnp.zeros((L,), jnp.float32)
        for d in range(D):
            col = jnp.full((L,), d, jnp.int32)
            gu = plsc.load_gather(urows_v, [rows, col])
            gm = plsc.load_gather(mrows_v, [rows, col])
            acc = acc + gu * gm
        outb_v[pl.ds(g * L, L)] = acc
        return carry

    lax.fori_loop(0, B_PER_W // L, body, 0)
    pltpu.sync_copy(outb_v, out_hbm.at[pl.ds(base, B_PER_W)])


def kernel(user_ids, movie_ids, user_table, movie_table):
    mesh = plsc.VectorSubcoreMesh(core_axis_name="c", subcore_axis_name="s")
    f = functools.partial(
        pl.kernel,
        mesh=mesh,
        out_type=jax.ShapeDtypeStruct((BATCH,), jnp.float32),
        scratch_types=[
            pltpu.VMEM((B_PER_W,), jnp.int32),
            pltpu.VMEM((B_PER_W,), jnp.int32),
            pltpu.VMEM((B_PER_W, D), jnp.float32),
            pltpu.VMEM((B_PER_W, D), jnp.float32),
            pltpu.VMEM((B_PER_W,), jnp.float32),
            pltpu.SemaphoreType.DMA,
            pltpu.SemaphoreType.DMA,
        ],
        compiler_params=pltpu.CompilerParams(
            use_tc_tiling_on_sc=False, needs_layout_passes=False),
    )(_mf_body)
    return f(user_ids.astype(jnp.int32), movie_ids.astype(jnp.int32),
             user_table, movie_table)
